# baseline (speedup 1.0000x reference)
import functools

import jax
import jax.numpy as jnp
from jax import lax
from jax.experimental import pallas as pl
from jax.experimental.pallas import tpu as pltpu
from jax.experimental.pallas import tpu_sc as plsc


def _enc1_body(tab_ref, w1_ref, b1_ref, out_ref):
    h = tab_ref[...]
    h1 = lax.dot_general(h, w1_ref[...], (((1,), (1,)), ((), ())),
                         preferred_element_type=jnp.float32) + b1_ref[...]
    out_ref[...] = h1 * jax.nn.sigmoid(h1)


def _enc1(table, W1, b1, row_block):
    V, D = table.shape
    return pl.pallas_call(
        _enc1_body,
        grid=(V // row_block,),
        in_specs=[
            pl.BlockSpec((row_block, D), lambda i: (i, 0)),
            pl.BlockSpec((D, D), lambda i: (0, 0)),
            pl.BlockSpec((1, D), lambda i: (0, 0)),
        ],
        out_specs=pl.BlockSpec((row_block, D), lambda i: (i, 0)),
        out_shape=jax.ShapeDtypeStruct((V, D), jnp.float32),
    )(table, W1, b1.reshape(1, D))


_CHUNK = 128
_NBUF = 2


def _make_gather(N, D, n_chunks_per_worker):
    mesh = plsc.VectorSubcoreMesh(core_axis_name="c", subcore_axis_name="s")
    nc = plsc.get_sparse_core_info().num_cores
    per_worker = n_chunks_per_worker * _CHUNK

    @functools.partial(
        pl.kernel,
        mesh=mesh,
        out_type=jax.ShapeDtypeStruct((N, D), jnp.float32),
        scratch_types=[
            pltpu.VMEM((n_chunks_per_worker, _CHUNK), jnp.int32),
            pltpu.VMEM((_CHUNK, D), jnp.float32),
            pltpu.VMEM((_CHUNK, D), jnp.float32),
            pltpu.SemaphoreType.DMA,
            pltpu.SemaphoreType.DMA,
        ],
    )
    def gather_kernel(tab_hbm, idx_hbm, out_hbm, idx_v, buf0, buf1,
                      sem0, sem1):
        wid = lax.axis_index("s") * nc + lax.axis_index("c")
        base = wid * per_worker
        pltpu.sync_copy(idx_hbm.at[wid], idx_v)

        bufs = (buf0, buf1)
        sems = (sem0, sem1)

        def start(j, slot):
            pltpu.async_copy(tab_hbm.at[idx_v.at[j]], bufs[slot], sems[slot])

        def finish(j, slot):
            pltpu.make_async_copy(tab_hbm.at[idx_v.at[j]], bufs[slot],
                                  sems[slot]).wait()
            pltpu.sync_copy(bufs[slot],
                            out_hbm.at[pl.ds(base + j * _CHUNK, _CHUNK)])

        for b in range(_NBUF):
            start(b, b)

        def body(i, carry):
            for b in range(_NBUF):
                j = i * _NBUF + b
                finish(j, b)
                nxt = j + _NBUF

                @pl.when(nxt < n_chunks_per_worker)
                def _():
                    start(nxt, b)
            return carry

        lax.fori_loop(0, n_chunks_per_worker // _NBUF, body, 0)

    return gather_kernel


def _l2_body(bb, S, g_ref, w2_ref, b2_ref, out_ref):
    h1 = g_ref[...]
    out = lax.dot_general(h1, w2_ref[...], (((1,), (1,)), ((), ())),
                          preferred_element_type=jnp.float32) + b2_ref[...]
    out_ref[...] = out.reshape(bb, S, out.shape[-1])


def _layer2(g, W2, b2, B, S, bb):
    D = g.shape[-1]
    return pl.pallas_call(
        functools.partial(_l2_body, bb, S),
        grid=(B // bb,),
        in_specs=[
            pl.BlockSpec((bb * S, D), lambda i: (i, 0)),
            pl.BlockSpec((D, D), lambda i: (0, 0)),
            pl.BlockSpec((1, D), lambda i: (0, 0)),
        ],
        out_specs=pl.BlockSpec((bb, S, D), lambda i: (i, 0, 0)),
        out_shape=jax.ShapeDtypeStruct((B, S, D), jnp.float32),
    )(g, W2, b2.reshape(1, D))


def kernel(x, table, W1, b1, W2, b2):
    B, S = x.shape
    V, D = table.shape
    N = B * S

    info = plsc.get_sparse_core_info()
    nw = info.num_cores * info.num_subcores

    assert N % (nw * _CHUNK) == 0
    n_chunks_per_worker = N // (nw * _CHUNK)
    assert n_chunks_per_worker % _NBUF == 0

    row_block = 2000
    assert V % row_block == 0
    h1t = _enc1(table, W1, b1, row_block)

    idx = x.reshape(nw, n_chunks_per_worker, _CHUNK).astype(jnp.int32)
    g = _make_gather(N, D, n_chunks_per_worker)(h1t, idx)

    bb = 32
    assert B % bb == 0
    return _layer2(g, W2, b2, B, S, bb)


# speedup vs baseline: 1.9856x; 1.9856x over previous
import functools

import jax
import jax.numpy as jnp
from jax import lax
from jax.experimental import pallas as pl
from jax.experimental.pallas import tpu as pltpu
from jax.experimental.pallas import tpu_sc as plsc


_CHUNK = 128
_NBUF = 2


def _make_gather(N, D, n_chunks_per_worker):
    mesh = plsc.VectorSubcoreMesh(core_axis_name="c", subcore_axis_name="s")
    nc = plsc.get_sparse_core_info().num_cores
    per_worker = n_chunks_per_worker * _CHUNK

    @functools.partial(
        pl.kernel,
        mesh=mesh,
        out_type=jax.ShapeDtypeStruct((N, D), jnp.float32),
        scratch_types=[
            pltpu.VMEM((n_chunks_per_worker, _CHUNK), jnp.int32),
            pltpu.VMEM((_CHUNK, D), jnp.float32),
            pltpu.VMEM((_CHUNK, D), jnp.float32),
            pltpu.SemaphoreType.DMA,
            pltpu.SemaphoreType.DMA,
        ],
    )
    def gather_kernel(tab_hbm, idx_hbm, out_hbm, idx_v, buf0, buf1,
                      sem0, sem1):
        wid = lax.axis_index("s") * nc + lax.axis_index("c")
        base = wid * per_worker
        pltpu.sync_copy(idx_hbm.at[wid], idx_v)

        bufs = (buf0, buf1)
        sems = (sem0, sem1)

        def start(j, slot):
            pltpu.async_copy(tab_hbm.at[idx_v.at[j]], bufs[slot], sems[slot])

        def finish(j, slot):
            pltpu.make_async_copy(tab_hbm.at[idx_v.at[j]], bufs[slot],
                                  sems[slot]).wait()
            pltpu.sync_copy(bufs[slot],
                            out_hbm.at[pl.ds(base + j * _CHUNK, _CHUNK)])

        for b in range(_NBUF):
            start(b, b)

        def body(i, carry):
            for b in range(_NBUF):
                j = i * _NBUF + b
                finish(j, b)
                nxt = j + _NBUF

                @pl.when(nxt < n_chunks_per_worker)
                def _():
                    start(nxt, b)
            return carry

        lax.fori_loop(0, n_chunks_per_worker // _NBUF, body, 0)

    return gather_kernel


def _mlp_body(B, g_ref, w1_ref, b1_ref, w2_ref, b2_ref, out_ref):
    h = g_ref[...]
    h1 = lax.dot_general(h, w1_ref[...], (((1,), (1,)), ((), ())),
                         preferred_element_type=jnp.float32) + b1_ref[...]
    h1 = h1 * jax.nn.sigmoid(h1)
    out = lax.dot_general(h1, w2_ref[...], (((1,), (1,)), ((), ())),
                          preferred_element_type=jnp.float32) + b2_ref[...]
    out_ref[...] = out.reshape(1, B, out.shape[-1])


def _mlp(g, W1, b1, W2, b2, B, S):
    D = g.shape[-1]
    return pl.pallas_call(
        functools.partial(_mlp_body, B),
        grid=(S,),
        in_specs=[
            pl.BlockSpec((B, D), lambda s: (s, 0)),
            pl.BlockSpec((D, D), lambda s: (0, 0)),
            pl.BlockSpec((1, D), lambda s: (0, 0)),
            pl.BlockSpec((D, D), lambda s: (0, 0)),
            pl.BlockSpec((1, D), lambda s: (0, 0)),
        ],
        out_specs=pl.BlockSpec((1, B, D), lambda s: (s, 0, 0)),
        out_shape=jax.ShapeDtypeStruct((S, B, D), jnp.float32),
    )(g, W1, b1.reshape(1, D), W2, b2.reshape(1, D))


def kernel(x, table, W1, b1, W2, b2):
    B, S = x.shape
    V, D = table.shape
    N = B * S

    info = plsc.get_sparse_core_info()
    nw = info.num_cores * info.num_subcores

    assert N % (nw * _CHUNK) == 0
    n_chunks_per_worker = N // (nw * _CHUNK)
    assert n_chunks_per_worker % _NBUF == 0

    idx = jnp.swapaxes(x, 0, 1).reshape(nw, n_chunks_per_worker, _CHUNK)
    idx = idx.astype(jnp.int32)
    g = _make_gather(N, D, n_chunks_per_worker)(table, idx)

    out_t = _mlp(g, W1, b1, W2, b2, B, S)
    return jnp.transpose(out_t, (1, 0, 2))


# speedup vs baseline: 2.3381x; 1.1775x over previous
import functools

import jax
import jax.numpy as jnp
from jax import lax
from jax.experimental import pallas as pl
from jax.experimental.pallas import tpu as pltpu
from jax.experimental.pallas import tpu_sc as plsc


def _encode_body(tab_ref, w1_ref, b1_ref, w2_ref, b2_ref, out_ref):
    h = tab_ref[...]
    h1 = lax.dot_general(h, w1_ref[...], (((1,), (1,)), ((), ())),
                         preferred_element_type=jnp.float32) + b1_ref[...]
    h1 = h1 * jax.nn.sigmoid(h1)
    out_ref[...] = lax.dot_general(h1, w2_ref[...], (((1,), (1,)), ((), ())),
                                   preferred_element_type=jnp.float32) + b2_ref[...]


def _encode_table(table, W1, b1, W2, b2, row_block):
    V, D = table.shape
    return pl.pallas_call(
        _encode_body,
        grid=(V // row_block,),
        in_specs=[
            pl.BlockSpec((row_block, D), lambda i: (i, 0)),
            pl.BlockSpec((D, D), lambda i: (0, 0)),
            pl.BlockSpec((1, D), lambda i: (0, 0)),
            pl.BlockSpec((D, D), lambda i: (0, 0)),
            pl.BlockSpec((1, D), lambda i: (0, 0)),
        ],
        out_specs=pl.BlockSpec((row_block, D), lambda i: (i, 0)),
        out_shape=jax.ShapeDtypeStruct((V, D), jnp.float32),
    )(table, W1, b1.reshape(1, D), W2, b2.reshape(1, D))


_CHUNK = 128
_NBUF = 2


def _make_gather(N, D, n_chunks_per_worker):
    mesh = plsc.VectorSubcoreMesh(core_axis_name="c", subcore_axis_name="s")
    nc = plsc.get_sparse_core_info().num_cores
    per_worker = n_chunks_per_worker * _CHUNK

    @functools.partial(
        pl.kernel,
        mesh=mesh,
        out_type=jax.ShapeDtypeStruct((N, D), jnp.float32),
        scratch_types=[
            pltpu.VMEM((n_chunks_per_worker, _CHUNK), jnp.int32),
            pltpu.VMEM((_CHUNK, D), jnp.float32),
            pltpu.VMEM((_CHUNK, D), jnp.float32),
            pltpu.SemaphoreType.DMA,
            pltpu.SemaphoreType.DMA,
        ],
    )
    def gather_kernel(tab_hbm, idx_hbm, out_hbm, idx_v, buf0, buf1,
                      sem0, sem1):
        wid = lax.axis_index("s") * nc + lax.axis_index("c")
        base = wid * per_worker
        pltpu.sync_copy(idx_hbm.at[wid], idx_v)

        bufs = (buf0, buf1)
        sems = (sem0, sem1)

        def start(j, slot):
            pltpu.async_copy(tab_hbm.at[idx_v.at[j]], bufs[slot], sems[slot])

        def finish(j, slot):
            pltpu.make_async_copy(tab_hbm.at[idx_v.at[j]], bufs[slot],
                                  sems[slot]).wait()
            pltpu.sync_copy(bufs[slot],
                            out_hbm.at[pl.ds(base + j * _CHUNK, _CHUNK)])

        for b in range(_NBUF):
            start(b, b)

        def body(i, carry):
            for b in range(_NBUF):
                j = i * _NBUF + b
                finish(j, b)
                nxt = j + _NBUF

                @pl.when(nxt < n_chunks_per_worker)
                def _():
                    start(nxt, b)
            return carry

        lax.fori_loop(0, n_chunks_per_worker // _NBUF, body, 0)

    return gather_kernel


def kernel(x, table, W1, b1, W2, b2):
    B, S = x.shape
    V, D = table.shape
    N = B * S

    info = plsc.get_sparse_core_info()
    nw = info.num_cores * info.num_subcores

    assert N % (nw * _CHUNK) == 0
    n_chunks_per_worker = N // (nw * _CHUNK)
    assert n_chunks_per_worker % _NBUF == 0

    row_block = 2000
    assert V % row_block == 0
    enc = _encode_table(table, W1, b1, W2, b2, row_block)

    idx = jnp.swapaxes(x, 0, 1).reshape(nw, n_chunks_per_worker, _CHUNK)
    idx = idx.astype(jnp.int32)
    g = _make_gather(N, D, n_chunks_per_worker)(enc, idx)

    return jnp.swapaxes(g.reshape(S, B, D), 0, 1)


# speedup vs baseline: 2.5918x; 1.1085x over previous
import functools

import jax
import jax.numpy as jnp
from jax import lax
from jax.experimental import pallas as pl
from jax.experimental.pallas import tpu as pltpu
from jax.experimental.pallas import tpu_sc as plsc


def _encode_body(tab_ref, w1_ref, b1_ref, w2_ref, b2_ref, out_ref):
    h = tab_ref[...]
    h1 = lax.dot_general(h, w1_ref[...], (((1,), (1,)), ((), ())),
                         preferred_element_type=jnp.float32) + b1_ref[...]
    h1 = h1 * jax.nn.sigmoid(h1)
    out_ref[...] = lax.dot_general(h1, w2_ref[...], (((1,), (1,)), ((), ())),
                                   preferred_element_type=jnp.float32) + b2_ref[...]


def _encode_table(table, W1, b1, W2, b2, row_block):
    V, D = table.shape
    return pl.pallas_call(
        _encode_body,
        grid=(V // row_block,),
        in_specs=[
            pl.BlockSpec((row_block, D), lambda i: (i, 0)),
            pl.BlockSpec((D, D), lambda i: (0, 0)),
            pl.BlockSpec((1, D), lambda i: (0, 0)),
            pl.BlockSpec((D, D), lambda i: (0, 0)),
            pl.BlockSpec((1, D), lambda i: (0, 0)),
        ],
        out_specs=pl.BlockSpec((row_block, D), lambda i: (i, 0)),
        out_shape=jax.ShapeDtypeStruct((V, D), jnp.float32),
    )(table, W1, b1.reshape(1, D), W2, b2.reshape(1, D))


_CHUNK = 128
_NBUF = 5


def _make_gather(N, D, n_chunks_per_worker):
    mesh = plsc.VectorSubcoreMesh(core_axis_name="c", subcore_axis_name="s")
    nc = plsc.get_sparse_core_info().num_cores
    per_worker = n_chunks_per_worker * _CHUNK

    @functools.partial(
        pl.kernel,
        mesh=mesh,
        out_type=jax.ShapeDtypeStruct((N, D), jnp.float32),
        scratch_types=(
            [pltpu.VMEM((n_chunks_per_worker, _CHUNK), jnp.int32)]
            + [pltpu.VMEM((_CHUNK, D), jnp.float32)] * _NBUF
            + [pltpu.SemaphoreType.DMA] * (2 * _NBUF)
        ),
    )
    def gather_kernel(tab_hbm, idx_hbm, out_hbm, idx_v, *bufs_sems):
        bufs = bufs_sems[:_NBUF]
        gsems = bufs_sems[_NBUF:2 * _NBUF]
        osems = bufs_sems[2 * _NBUF:]
        wid = lax.axis_index("s") * nc + lax.axis_index("c")
        base = wid * per_worker
        pltpu.sync_copy(idx_hbm.at[wid], idx_v)

        def out_slice(j):
            return out_hbm.at[pl.ds(base + j * _CHUNK, _CHUNK)]

        def start_gather(j, slot):
            pltpu.async_copy(tab_hbm.at[idx_v.at[j]], bufs[slot], gsems[slot])

        for b in range(_NBUF):
            start_gather(b, b)

        def body(i, carry):
            for b in range(_NBUF):
                j = i * _NBUF + b
                pltpu.make_async_copy(tab_hbm.at[idx_v.at[j]], bufs[b],
                                      gsems[b]).wait()
                pltpu.async_copy(bufs[b], out_slice(j), osems[b])
                nxt = j + _NBUF

                @pl.when(nxt < n_chunks_per_worker)
                def _():
                    pltpu.make_async_copy(bufs[b], out_slice(j),
                                          osems[b]).wait()
                    start_gather(nxt, b)
            return carry

        lax.fori_loop(0, n_chunks_per_worker // _NBUF, body, 0)

        for b in range(_NBUF):
            j = n_chunks_per_worker - _NBUF + b
            pltpu.make_async_copy(bufs[b], out_slice(j), osems[b]).wait()

    return gather_kernel


def kernel(x, table, W1, b1, W2, b2):
    B, S = x.shape
    V, D = table.shape
    N = B * S

    info = plsc.get_sparse_core_info()
    nw = info.num_cores * info.num_subcores

    assert N % (nw * _CHUNK) == 0
    n_chunks_per_worker = N // (nw * _CHUNK)
    assert n_chunks_per_worker % _NBUF == 0

    row_block = 5000
    assert V % row_block == 0
    enc = _encode_table(table, W1, b1, W2, b2, row_block)

    idx = jnp.swapaxes(x, 0, 1).reshape(nw, n_chunks_per_worker, _CHUNK)
    idx = idx.astype(jnp.int32)
    g = _make_gather(N, D, n_chunks_per_worker)(enc, idx)

    return jnp.swapaxes(g.reshape(S, B, D), 0, 1)


# speedup vs baseline: 2.8076x; 1.0832x over previous
import functools

import jax
import jax.numpy as jnp
from jax import lax
from jax.experimental import pallas as pl
from jax.experimental.pallas import tpu as pltpu
from jax.experimental.pallas import tpu_sc as plsc


def _encode_body(tab_ref, w1_ref, b1_ref, w2_ref, b2_ref, out_ref):
    h = tab_ref[...]
    h1 = lax.dot_general(h, w1_ref[...], (((1,), (1,)), ((), ())),
                         preferred_element_type=jnp.float32) + b1_ref[...]
    h1 = h1 * jax.nn.sigmoid(h1)
    out_ref[...] = lax.dot_general(h1, w2_ref[...], (((1,), (1,)), ((), ())),
                                   preferred_element_type=jnp.float32) + b2_ref[...]


def _encode_table(table, W1, b1, W2, b2, row_block):
    V, D = table.shape
    return pl.pallas_call(
        _encode_body,
        grid=(V // row_block,),
        in_specs=[
            pl.BlockSpec((row_block, D), lambda i: (i, 0)),
            pl.BlockSpec((D, D), lambda i: (0, 0)),
            pl.BlockSpec((1, D), lambda i: (0, 0)),
            pl.BlockSpec((D, D), lambda i: (0, 0)),
            pl.BlockSpec((1, D), lambda i: (0, 0)),
        ],
        out_specs=pl.BlockSpec((row_block, D), lambda i: (i, 0)),
        out_shape=jax.ShapeDtypeStruct((V, D), jnp.float32),
    )(table, W1, b1.reshape(1, D), W2, b2.reshape(1, D))


_CHUNK = 128
_NBUF = 5


def _make_gather(N, D, n_chunks_per_worker):
    mesh = plsc.VectorSubcoreMesh(core_axis_name="c", subcore_axis_name="s")
    nc = plsc.get_sparse_core_info().num_cores
    per_worker = n_chunks_per_worker * _CHUNK

    @functools.partial(
        pl.kernel,
        mesh=mesh,
        out_type=jax.ShapeDtypeStruct((N, D), jnp.float32),
        scratch_types=(
            [pltpu.VMEM((n_chunks_per_worker, _CHUNK), jnp.int32)]
            + [pltpu.VMEM((_CHUNK, D), jnp.float32)] * _NBUF
            + [pltpu.SemaphoreType.DMA] * (2 * _NBUF)
        ),
    )
    def gather_kernel(tab_hbm, idx_hbm, out_hbm, idx_v, *bufs_sems):
        bufs = bufs_sems[:_NBUF]
        gsems = bufs_sems[_NBUF:2 * _NBUF]
        osems = bufs_sems[2 * _NBUF:]
        wid = lax.axis_index("s") * nc + lax.axis_index("c")
        base = wid * per_worker
        pltpu.sync_copy(idx_hbm.at[wid], idx_v)

        def out_slice(j):
            return out_hbm.at[pl.ds(base + j * _CHUNK, _CHUNK)]

        def start_gather(j, slot):
            pltpu.async_copy(tab_hbm.at[idx_v.at[j]], bufs[slot], gsems[slot])

        for b in range(_NBUF):
            start_gather(b, b)

        def body(i, carry):
            for b in range(_NBUF):
                j = i * _NBUF + b
                pltpu.make_async_copy(tab_hbm.at[idx_v.at[j]], bufs[b],
                                      gsems[b]).wait()
                pltpu.async_copy(bufs[b], out_slice(j), osems[b])
                nxt = j + _NBUF

                @pl.when(nxt < n_chunks_per_worker)
                def _():
                    pltpu.make_async_copy(bufs[b], out_slice(j),
                                          osems[b]).wait()
                    start_gather(nxt, b)
            return carry

        lax.fori_loop(0, n_chunks_per_worker // _NBUF, body, 0)

        for b in range(_NBUF):
            j = n_chunks_per_worker - _NBUF + b
            pltpu.make_async_copy(bufs[b], out_slice(j), osems[b]).wait()

    return gather_kernel


def kernel(x, table, W1, b1, W2, b2):
    B, S = x.shape
    V, D = table.shape
    N = B * S

    info = plsc.get_sparse_core_info()
    nw = info.num_cores * info.num_subcores

    assert N % (nw * _CHUNK) == 0
    n_chunks_per_worker = N // (nw * _CHUNK)
    assert n_chunks_per_worker % _NBUF == 0

    row_block = 10000
    assert V % row_block == 0
    enc = _encode_table(table, W1, b1, W2, b2, row_block)

    idx = jnp.swapaxes(x, 0, 1).reshape(nw, n_chunks_per_worker, _CHUNK)
    idx = idx.astype(jnp.int32)
    g = _make_gather(N, D, n_chunks_per_worker)(enc, idx)

    return jnp.swapaxes(g.reshape(S, B, D), 0, 1)


# speedup vs baseline: 2.8131x; 1.0020x over previous
import functools

import jax
import jax.numpy as jnp
from jax import lax
from jax.experimental import pallas as pl
from jax.experimental.pallas import tpu as pltpu
from jax.experimental.pallas import tpu_sc as plsc


def _encode_body(tab_ref, w1_ref, b1_ref, w2_ref, b2_ref, out_ref):
    h = tab_ref[...]
    h1 = lax.dot_general(h, w1_ref[...], (((1,), (1,)), ((), ())),
                         preferred_element_type=jnp.float32) + b1_ref[...]
    h1 = h1 * jax.nn.sigmoid(h1)
    out_ref[...] = lax.dot_general(h1, w2_ref[...], (((1,), (1,)), ((), ())),
                                   preferred_element_type=jnp.float32) + b2_ref[...]


def _encode_table(table, W1, b1, W2, b2, row_block):
    V, D = table.shape
    return pl.pallas_call(
        _encode_body,
        grid=(V // row_block,),
        in_specs=[
            pl.BlockSpec((row_block, D), lambda i: (i, 0)),
            pl.BlockSpec((D, D), lambda i: (0, 0)),
            pl.BlockSpec((1, D), lambda i: (0, 0)),
            pl.BlockSpec((D, D), lambda i: (0, 0)),
            pl.BlockSpec((1, D), lambda i: (0, 0)),
        ],
        out_specs=pl.BlockSpec((row_block, D), lambda i: (i, 0)),
        out_shape=jax.ShapeDtypeStruct((V, D), jnp.float32),
    )(table, W1, b1.reshape(1, D), W2, b2.reshape(1, D))


_CHUNK = 128
_NBUF = 5


def _make_gather(N, D, n_chunks_per_worker):
    mesh = plsc.VectorSubcoreMesh(core_axis_name="c", subcore_axis_name="s")
    nc = plsc.get_sparse_core_info().num_cores
    per_worker = n_chunks_per_worker * _CHUNK

    @functools.partial(
        pl.kernel,
        mesh=mesh,
        out_type=jax.ShapeDtypeStruct((N, D), jnp.float32),
        scratch_types=(
            [pltpu.VMEM((n_chunks_per_worker, _CHUNK), jnp.int32)]
            + [pltpu.VMEM((_CHUNK, D), jnp.float32)] * _NBUF
            + [pltpu.SemaphoreType.DMA] * (2 * _NBUF)
        ),
    )
    def gather_kernel(tab_hbm, idx_hbm, out_hbm, idx_v, *bufs_sems):
        bufs = bufs_sems[:_NBUF]
        gsems = bufs_sems[_NBUF:2 * _NBUF]
        osems = bufs_sems[2 * _NBUF:]
        wid = lax.axis_index("s") * nc + lax.axis_index("c")
        base = wid * per_worker
        pltpu.sync_copy(idx_hbm.at[wid], idx_v)

        def out_slice(j):
            return out_hbm.at[pl.ds(base + j * _CHUNK, _CHUNK)]

        def start_gather(j, slot):
            pltpu.async_copy(tab_hbm.at[idx_v.at[j]], bufs[slot], gsems[slot])

        for b in range(_NBUF):
            start_gather(b, b)

        def body(i, carry):
            for b in range(_NBUF):
                j = i * _NBUF + b
                pltpu.make_async_copy(tab_hbm.at[idx_v.at[j]], bufs[b],
                                      gsems[b]).wait()
                pltpu.async_copy(bufs[b], out_slice(j), osems[b])
                nxt = j + _NBUF

                @pl.when(nxt < n_chunks_per_worker)
                def _():
                    pltpu.make_async_copy(bufs[b], out_slice(j),
                                          osems[b]).wait()
                    start_gather(nxt, b)
            return carry

        lax.fori_loop(0, n_chunks_per_worker // _NBUF, body, 0)

        for b in range(_NBUF):
            j = n_chunks_per_worker - _NBUF + b
            pltpu.make_async_copy(bufs[b], out_slice(j), osems[b]).wait()

    return gather_kernel


def kernel(x, table, W1, b1, W2, b2):
    B, S = x.shape
    V, D = table.shape
    N = B * S

    info = plsc.get_sparse_core_info()
    nw = info.num_cores * info.num_subcores

    assert N % (nw * _CHUNK) == 0
    n_chunks_per_worker = N // (nw * _CHUNK)
    assert n_chunks_per_worker % _NBUF == 0

    row_block = 20000
    assert V % row_block == 0
    enc = _encode_table(table, W1, b1, W2, b2, row_block)

    idx = jnp.swapaxes(x, 0, 1).reshape(nw, n_chunks_per_worker, _CHUNK)
    idx = idx.astype(jnp.int32)
    g = _make_gather(N, D, n_chunks_per_worker)(enc, idx)

    return jnp.swapaxes(g.reshape(S, B, D), 0, 1)


# speedup vs baseline: 2.8140x; 1.0003x over previous
import functools

import jax
import jax.numpy as jnp
from jax import lax
from jax.experimental import pallas as pl
from jax.experimental.pallas import tpu as pltpu
from jax.experimental.pallas import tpu_sc as plsc


def _encode_body(tab_ref, w1_ref, b1_ref, w2_ref, b2_ref, out_ref):
    h = tab_ref[...]
    h1 = lax.dot_general(h, w1_ref[...], (((1,), (1,)), ((), ())),
                         preferred_element_type=jnp.float32) + b1_ref[...]
    h1 = h1 * jax.nn.sigmoid(h1)
    out_ref[...] = lax.dot_general(h1, w2_ref[...], (((1,), (1,)), ((), ())),
                                   preferred_element_type=jnp.float32) + b2_ref[...]


def _encode_table(table, W1, b1, W2, b2, row_block):
    V, D = table.shape
    return pl.pallas_call(
        _encode_body,
        grid=(V // row_block,),
        in_specs=[
            pl.BlockSpec((row_block, D), lambda i: (i, 0)),
            pl.BlockSpec((D, D), lambda i: (0, 0)),
            pl.BlockSpec((1, D), lambda i: (0, 0)),
            pl.BlockSpec((D, D), lambda i: (0, 0)),
            pl.BlockSpec((1, D), lambda i: (0, 0)),
        ],
        out_specs=pl.BlockSpec((row_block, D), lambda i: (i, 0)),
        out_shape=jax.ShapeDtypeStruct((V, D), jnp.float32),
    )(table, W1, b1.reshape(1, D), W2, b2.reshape(1, D))


_CHUNK = 64
_NBUF = 10


def _make_gather(N, D, n_chunks_per_worker):
    mesh = plsc.VectorSubcoreMesh(core_axis_name="c", subcore_axis_name="s")
    nc = plsc.get_sparse_core_info().num_cores
    per_worker = n_chunks_per_worker * _CHUNK

    @functools.partial(
        pl.kernel,
        mesh=mesh,
        out_type=jax.ShapeDtypeStruct((N, D), jnp.float32),
        scratch_types=(
            [pltpu.VMEM((n_chunks_per_worker, _CHUNK), jnp.int32)]
            + [pltpu.VMEM((_CHUNK, D), jnp.float32)] * _NBUF
            + [pltpu.SemaphoreType.DMA] * (2 * _NBUF)
        ),
    )
    def gather_kernel(tab_hbm, idx_hbm, out_hbm, idx_v, *bufs_sems):
        bufs = bufs_sems[:_NBUF]
        gsems = bufs_sems[_NBUF:2 * _NBUF]
        osems = bufs_sems[2 * _NBUF:]
        wid = lax.axis_index("s") * nc + lax.axis_index("c")
        base = wid * per_worker
        pltpu.sync_copy(idx_hbm.at[wid], idx_v)

        def out_slice(j):
            return out_hbm.at[pl.ds(base + j * _CHUNK, _CHUNK)]

        def start_gather(j, slot):
            pltpu.async_copy(tab_hbm.at[idx_v.at[j]], bufs[slot], gsems[slot])

        for b in range(_NBUF):
            start_gather(b, b)

        def body(i, carry):
            for b in range(_NBUF):
                j = i * _NBUF + b
                pltpu.make_async_copy(tab_hbm.at[idx_v.at[j]], bufs[b],
                                      gsems[b]).wait()
                pltpu.async_copy(bufs[b], out_slice(j), osems[b])
                nxt = j + _NBUF

                @pl.when(nxt < n_chunks_per_worker)
                def _():
                    pltpu.make_async_copy(bufs[b], out_slice(j),
                                          osems[b]).wait()
                    start_gather(nxt, b)
            return carry

        lax.fori_loop(0, n_chunks_per_worker // _NBUF, body, 0)

        for b in range(_NBUF):
            j = n_chunks_per_worker - _NBUF + b
            pltpu.make_async_copy(bufs[b], out_slice(j), osems[b]).wait()

    return gather_kernel


def kernel(x, table, W1, b1, W2, b2):
    B, S = x.shape
    V, D = table.shape
    N = B * S

    info = plsc.get_sparse_core_info()
    nw = info.num_cores * info.num_subcores

    assert N % (nw * _CHUNK) == 0
    n_chunks_per_worker = N // (nw * _CHUNK)
    assert n_chunks_per_worker % _NBUF == 0

    row_block = 20000
    assert V % row_block == 0
    enc = _encode_table(table, W1, b1, W2, b2, row_block)

    idx = jnp.swapaxes(x, 0, 1).reshape(nw, n_chunks_per_worker, _CHUNK)
    idx = idx.astype(jnp.int32)
    g = _make_gather(N, D, n_chunks_per_worker)(enc, idx)

    return jnp.swapaxes(g.reshape(S, B, D), 0, 1)
